# Initial kernel scaffold; baseline (speedup 1.0000x reference)
import jax
import jax.numpy as jnp
from jax.experimental import pallas as pl


def kernel(x, edge_index, W1, b1, W2, b2):
    raise NotImplementedError("write your pallas kernel here")


# speedup vs baseline: 1.1645x; 1.1645x over previous
import jax
import jax.numpy as jnp
from jax import lax
from jax.experimental import pallas as pl

_N = 10000
_D = 128
_N_PAD = 10240
_TC_BLK = 1024


def _norm(deg_blk, which):
    return lax.rsqrt(jnp.clip(deg_blk[which], 1.0, None))[:, None]


def _prep_tc(x_ref, w_ref, deg_ref, o_ref):
    xw = jnp.dot(x_ref[...], w_ref[...], preferred_element_type=jnp.float32)
    o_ref[...] = xw * _norm(deg_ref[...], 0)


def _mid_tc(p_ref, deg_ref, b1_ref, w2_ref, o_ref):
    h = jnp.maximum(p_ref[...] * _norm(deg_ref[...], 1) + b1_ref[...], 0.0)
    hw = jnp.dot(h, w2_ref[...], preferred_element_type=jnp.float32)
    o_ref[...] = hw * _norm(deg_ref[...], 0)


def _final_tc(p_ref, deg_ref, b2_ref, o_ref):
    o_ref[...] = p_ref[...] * _norm(deg_ref[...], 1) + b2_ref[...]


_GRID = (_N_PAD // _TC_BLK,)
_deg_spec = pl.BlockSpec((2, _TC_BLK), lambda i: (0, i))
_row_spec = pl.BlockSpec((_TC_BLK, _D), lambda i: (i, 0))
_mat_spec = pl.BlockSpec((_D, _D), lambda i: (0, 0))
_bias_spec = pl.BlockSpec((1, _D), lambda i: (0, 0))
_out_struct = jax.ShapeDtypeStruct((_N_PAD, _D), jnp.float32)


def kernel(x, edge_index, W1, b1, W2, b2):
    src = edge_index[0]
    dst = edge_index[1]

    deg = jnp.stack([
        jnp.bincount(src, length=_N_PAD).astype(jnp.float32),
        jnp.bincount(dst, length=_N_PAD).astype(jnp.float32),
    ])
    x_p = jnp.pad(x, ((0, _N_PAD - _N), (0, 0)))
    b1r = b1.reshape(1, _D)
    b2r = b2.reshape(1, _D)

    def _agg(z):
        return jnp.zeros((_N_PAD, _D), jnp.float32).at[dst].add(z[src])

    z1 = pl.pallas_call(
        _prep_tc,
        grid=_GRID,
        in_specs=[_row_spec, _mat_spec, _deg_spec],
        out_specs=_row_spec,
        out_shape=_out_struct,
    )(x_p, W1, deg)

    agg1 = _agg(z1)

    z2 = pl.pallas_call(
        _mid_tc,
        grid=_GRID,
        in_specs=[_row_spec, _deg_spec, _bias_spec, _mat_spec],
        out_specs=_row_spec,
        out_shape=_out_struct,
    )(agg1, deg, b1r, W2)

    agg2 = _agg(z2)

    out = pl.pallas_call(
        _final_tc,
        grid=_GRID,
        in_specs=[_row_spec, _deg_spec, _bias_spec],
        out_specs=_row_spec,
        out_shape=_out_struct,
    )(agg2, deg, b2r)

    return out[:_N]
